# Initial kernel scaffold; baseline (speedup 1.0000x reference)
import jax
import jax.numpy as jnp
from jax.experimental import pallas as pl


def kernel(users, items, edge_index, user_emb, item_emb):
    raise NotImplementedError("write your pallas kernel here")


# speedup vs baseline: 9.6983x; 9.6983x over previous
import functools

import jax
import jax.numpy as jnp
from jax import lax
from jax.experimental import pallas as pl
from jax.experimental.pallas import tpu as pltpu
from jax.experimental.pallas import tpu_sc as plsc

NU = 25000
NN = 50000
D = 64
E = 800000
BATCH = 16384

NC, NS = 2, 16
NW = NC * NS
HALF = 25000
ACC_ROWS = 25088
NN_PAD = 50176
ER = E // 128
ER_PAD = 6272
E_PAD = ER_PAD * 128
EPT_ROWS = ER_PAD // NS
N_EDGE_BLK = EPT_ROWS // 8
ROW_BLOCKS = 196
LAST_ROW_START = HALF - 128
DEG_HALF = NN_PAD // 2
DEG_STRIP = DEG_HALF // NS

_mesh = plsc.VectorSubcoreMesh(
    core_axis_name="c", subcore_axis_name="s", num_cores=NC, num_subcores=NS)

_f32 = jnp.float32
_i32 = jnp.int32


def _zero_fill(ref, rows):
    z = jnp.zeros((16,), _f32)

    def body(r, _):
        for q in range(4):
            ref[r, pl.ds(q * 16, 16)] = z
        return 0

    lax.fori_loop(0, rows, body, 0)


@functools.partial(
    pl.kernel,
    out_type=(
        jax.ShapeDtypeStruct((ER_PAD, 128), _i32),
        jax.ShapeDtypeStruct((ER_PAD, 128), _i32),
    ),
    mesh=_mesh,
    compiler_params=pltpu.CompilerParams(use_tc_tiling_on_sc=False, needs_layout_passes=False),
    scratch_types=[
        pltpu.VMEM((8, 128), _i32),
        pltpu.VMEM((8, 128), _i32),
        pltpu.VMEM((8, 128), _i32),
    ],
)
def _transform_edges(dst_h, d0p, d1p, db, b0, b1):
    c = lax.axis_index("c")
    s = lax.axis_index("s")
    wid = c * NS + s
    nb = ER_PAD // 8
    lane = lax.iota(_i32, 16)

    def body(j, _):
        g = j * NW + wid

        @pl.when(g < nb)
        def _():
            r = g * 8
            pltpu.sync_copy(dst_h.at[pl.ds(r, 8)], db)
            for i in range(8):
                for q in range(8):
                    sl = (i, pl.ds(q * 16, 16))
                    dv = db[sl]
                    dummy = HALF + lane
                    b0[sl] = jnp.where(dv < HALF, dv, dummy)
                    b1[sl] = jnp.where(dv >= HALF, dv - HALF, dummy)
            pltpu.sync_copy(b0, d0p.at[pl.ds(r, 8)])
            pltpu.sync_copy(b1, d1p.at[pl.ds(r, 8)])
        return 0

    lax.fori_loop(0, (nb + NW - 1) // NW, body, 0)


@functools.partial(
    pl.kernel,
    out_type=(
        jax.ShapeDtypeStruct((NN_PAD,), _f32),
        jax.ShapeDtypeStruct((NN_PAD,), _f32),
    ),
    mesh=_mesh,
    compiler_params=pltpu.CompilerParams(use_tc_tiling_on_sc=False, needs_layout_passes=False),
    scratch_types=[
        pltpu.VMEM_SHARED((NN_PAD,), _f32),
        pltpu.VMEM_SHARED((NN_PAD,), _f32),
        pltpu.VMEM((DEG_STRIP,), _f32),
        pltpu.VMEM((8, 128), _i32),
        pltpu.VMEM((128,), _f32),
    ],
)
def _hist(srcp, dstp, degs_h, degd_h, deg_s, deg_d, zb, ib, ones):
    c = lax.axis_index("c")
    s = lax.axis_index("s")

    def zfill(i, _):
        zb[pl.ds(i * 16, 16)] = jnp.zeros((16,), _f32)
        return 0

    lax.fori_loop(0, DEG_STRIP // 16, zfill, 0)
    for q in range(8):
        ones[pl.ds(q * 16, 16)] = jnp.ones((16,), _f32)
    for h in range(2):
        strip = (h * NS + s) * DEG_STRIP
        pltpu.sync_copy(zb, deg_s.at[pl.ds(strip, DEG_STRIP)])
        pltpu.sync_copy(zb, deg_d.at[pl.ds(strip, DEG_STRIP)])
    plsc.subcore_barrier()

    def hist(g, _):
        r = s * EPT_ROWS + g * 8
        pltpu.sync_copy(srcp.at[pl.ds(r, 8)], ib)
        for j in range(8):
            pltpu.sync_copy(ones, deg_s.at[ib.at[j]], add=True)
        pltpu.sync_copy(dstp.at[pl.ds(r, 8)], ib)
        for j in range(8):
            pltpu.sync_copy(ones, deg_d.at[ib.at[j]], add=True)
        return 0

    lax.fori_loop(0, N_EDGE_BLK, hist, 0)
    plsc.subcore_barrier()

    base = c * DEG_HALF + s * DEG_STRIP
    pltpu.sync_copy(deg_s.at[pl.ds(base, DEG_STRIP)], zb)
    pltpu.sync_copy(zb, degs_h.at[pl.ds(base, DEG_STRIP)])
    pltpu.sync_copy(deg_d.at[pl.ds(base, DEG_STRIP)], zb)
    pltpu.sync_copy(zb, degd_h.at[pl.ds(base, DEG_STRIP)])


_TC_BLK = 1024
_TC_GRID = NN_PAD // _TC_BLK


def _prep_tc(deg_s, deg_d, t0):
    def body(ds_ref, dd_ref, t0_ref, win_ref, wprod_ref, t0p_ref):
        iso = lax.rsqrt(jnp.maximum(ds_ref[...], 1.0))
        isi = lax.rsqrt(jnp.maximum(dd_ref[...], 1.0))
        win_ref[...] = isi
        wprod_ref[...] = isi * iso
        t0p_ref[...] = t0_ref[...] * iso

    return pl.pallas_call(
        body,
        grid=(_TC_GRID,),
        in_specs=[
            pl.BlockSpec((_TC_BLK, 1), lambda i: (i, 0)),
            pl.BlockSpec((_TC_BLK, 1), lambda i: (i, 0)),
            pl.BlockSpec((_TC_BLK, D), lambda i: (i, 0)),
        ],
        out_specs=[
            pl.BlockSpec((_TC_BLK, 1), lambda i: (i, 0)),
            pl.BlockSpec((_TC_BLK, 1), lambda i: (i, 0)),
            pl.BlockSpec((_TC_BLK, D), lambda i: (i, 0)),
        ],
        out_shape=[
            jax.ShapeDtypeStruct((NN_PAD, 1), _f32),
            jax.ShapeDtypeStruct((NN_PAD, 1), _f32),
            jax.ShapeDtypeStruct((NN_PAD, D), _f32),
        ],
    )(deg_s, deg_d, t0)


def _scale_tc(s_tab, win, wprod):
    def body(s_ref, win_ref, wprod_ref, t_ref, tp_ref):
        sv = s_ref[...]
        t_ref[...] = sv * win_ref[...]
        tp_ref[...] = sv * wprod_ref[...]

    return pl.pallas_call(
        body,
        grid=(_TC_GRID,),
        in_specs=[
            pl.BlockSpec((_TC_BLK, D), lambda i: (i, 0)),
            pl.BlockSpec((_TC_BLK, 1), lambda i: (i, 0)),
            pl.BlockSpec((_TC_BLK, 1), lambda i: (i, 0)),
        ],
        out_specs=[
            pl.BlockSpec((_TC_BLK, D), lambda i: (i, 0)),
            pl.BlockSpec((_TC_BLK, D), lambda i: (i, 0)),
        ],
        out_shape=[
            jax.ShapeDtypeStruct((NN_PAD, D), _f32),
            jax.ShapeDtypeStruct((NN_PAD, D), _f32),
        ],
    )(s_tab, win, wprod)


@functools.partial(
    pl.kernel,
    out_type=jax.ShapeDtypeStruct((NN_PAD, D), _f32),
    mesh=_mesh,
    compiler_params=pltpu.CompilerParams(use_tc_tiling_on_sc=False, needs_layout_passes=False),
    scratch_types=[
        pltpu.VMEM_SHARED((ACC_ROWS, D), _f32),
        pltpu.VMEM((8, 128), _i32),
        pltpu.VMEM((8, 128), _i32),
        pltpu.VMEM((256, D), _f32),
        pltpu.SemaphoreType.DMA,
    ],
)
def _layer(tp_in, srcp, d0p, d1p, s_out, acc, sb, db, rows, sem):
    c = lax.axis_index("c")
    s = lax.axis_index("s")

    _zero_fill(rows, 128)
    zbase = s * (ACC_ROWS // NS)

    def zloop(k, _):
        start = zbase + jnp.minimum(k * 128, ACC_ROWS // NS - 128)
        pltpu.sync_copy(rows.at[pl.ds(0, 128)], acc.at[pl.ds(start, 128)])
        return 0

    lax.fori_loop(0, 13, zloop, 0)
    plsc.subcore_barrier()

    def edges(g, _):
        r = s * EPT_ROWS + g * 8
        pltpu.sync_copy(srcp.at[pl.ds(r, 8)], sb)

        @pl.when(c == 0)
        def _():
            pltpu.sync_copy(d0p.at[pl.ds(r, 8)], db)

        @pl.when(c == 1)
        def _():
            pltpu.sync_copy(d1p.at[pl.ds(r, 8)], db)

        def buf(j):
            return rows.at[pl.ds((j % 2) * 128, 128)]

        cp = pltpu.async_copy(tp_in.at[sb.at[0]], buf(0), sem)
        for j in range(8):
            cp.wait()
            if j < 7:
                cp = pltpu.async_copy(tp_in.at[sb.at[j + 1]], buf(j + 1), sem)
            pltpu.sync_copy(buf(j), acc.at[db.at[j]], add=True)
        return 0

    lax.fori_loop(0, N_EDGE_BLK, edges, 0)
    plsc.subcore_barrier()

    def rows_out(j, _):
        g = j * NS + s

        @pl.when(g < ROW_BLOCKS)
        def _():
            lstart = jnp.minimum(g * 128, LAST_ROW_START)
            gstart = c * HALF + lstart
            pltpu.sync_copy(acc.at[pl.ds(lstart, 128)], rows.at[pl.ds(0, 128)])
            pltpu.sync_copy(rows.at[pl.ds(0, 128)], s_out.at[pl.ds(gstart, 128)])
        return 0

    lax.fori_loop(0, 13, rows_out, 0)


@functools.partial(
    pl.kernel,
    out_type=jax.ShapeDtypeStruct((BATCH,), _f32),
    mesh=_mesh,
    compiler_params=pltpu.CompilerParams(use_tc_tiling_on_sc=False, needs_layout_passes=False),
    scratch_types=[
        pltpu.VMEM((128,), _i32),
        pltpu.VMEM((128,), _i32),
        pltpu.VMEM((128, D), _f32), pltpu.VMEM((128, D), _f32),
        pltpu.VMEM((128, D), _f32), pltpu.VMEM((128, D), _f32),
        pltpu.VMEM((128, D), _f32), pltpu.VMEM((128, D), _f32),
        pltpu.VMEM((128, D), _f32), pltpu.VMEM((128, D), _f32),
        pltpu.VMEM((128,), _f32),
        pltpu.SemaphoreType.DMA,
    ],
)
def _final(users_h, items_h, t0, t1, t2, t3, gamma_h,
           ub, vb, u0, u1, u2, u3, i0, i1, i2, i3, gb, sem):
    c = lax.axis_index("c")
    s = lax.axis_index("s")
    wid = c * NS + s

    def block(t, _):
        sb = wid * 4 + t
        pltpu.sync_copy(users_h.at[pl.ds(sb * 128, 128)], ub)
        pltpu.sync_copy(items_h.at[pl.ds(sb * 128, 128)], vb)
        for q in range(8):
            sl = pl.ds(q * 16, 16)
            vb[sl] = vb[sl] + NU
        cps = [
            pltpu.async_copy(t0.at[ub], u0, sem),
            pltpu.async_copy(t1.at[ub], u1, sem),
            pltpu.async_copy(t2.at[ub], u2, sem),
            pltpu.async_copy(t3.at[ub], u3, sem),
            pltpu.async_copy(t0.at[vb], i0, sem),
            pltpu.async_copy(t1.at[vb], i1, sem),
            pltpu.async_copy(t2.at[vb], i2, sem),
            pltpu.async_copy(t3.at[vb], i3, sem),
        ]
        for cp in cps:
            cp.wait()

        lane = lax.iota(_i32, 16)

        def dot(rb, _):
            res = jnp.zeros((16,), _f32)
            for rr in range(16):
                r = rb * 16 + rr
                acc = jnp.zeros((16,), _f32)
                for q in range(4):
                    sl = (r, pl.ds(q * 16, 16))
                    fu = (u0[sl] + u1[sl]) + (u2[sl] + u3[sl])
                    fi = (i0[sl] + i1[sl]) + (i2[sl] + i3[sl])
                    acc = acc + fu * fi
                res = res + jnp.where(lane == rr, jnp.sum(acc), 0.0)
            gb[pl.ds(rb * 16, 16)] = res * 0.0625
            return 0

        lax.fori_loop(0, 8, dot, 0)
        pltpu.sync_copy(gb, gamma_h.at[pl.ds(sb * 128, 128)])
        return 0

    lax.fori_loop(0, 4, block, 0)


def kernel(users, items, edge_index, user_emb, item_emb):
    srcp = jnp.pad(edge_index[0], (0, E_PAD - E),
                   constant_values=NN).reshape(ER_PAD, 128)
    dstp = jnp.pad(edge_index[1], (0, E_PAD - E),
                   constant_values=NN).reshape(ER_PAD, 128)
    t0 = jnp.concatenate([user_emb, item_emb], axis=0)
    t0 = jnp.pad(t0, ((0, NN_PAD - NN), (0, 0)))

    d0p, d1p = _transform_edges(dstp)
    deg_s, deg_d = _hist(srcp, dstp)
    win, wprod, tp = _prep_tc(
        deg_s.reshape(NN_PAD, 1), deg_d.reshape(NN_PAD, 1), t0)
    s1 = _layer(tp, srcp, d0p, d1p)
    t1, tp = _scale_tc(s1, win, wprod)
    s2 = _layer(tp, srcp, d0p, d1p)
    t2, tp = _scale_tc(s2, win, wprod)
    s3 = _layer(tp, srcp, d0p, d1p)
    t3, _ = _scale_tc(s3, win, wprod)

    return _final(users, items, t0, t1, t2, t3)


# speedup vs baseline: 12.9514x; 1.3354x over previous
import functools

import jax
import jax.numpy as jnp
from jax import lax
from jax.experimental import pallas as pl
from jax.experimental.pallas import tpu as pltpu
from jax.experimental.pallas import tpu_sc as plsc

NU = 25000
NN = 50000
D = 64
E = 800000
BATCH = 16384

NC, NS = 2, 16
NW = NC * NS
HALF = 25000
ACC_ROWS = 25088
NN_PAD = 50176
ER = E // 128
ER_PAD = 6272
E_PAD = ER_PAD * 128
EPT_ROWS = ER_PAD // NS
N_EDGE_BLK = EPT_ROWS // 8
ROW_BLOCKS = 196
LAST_ROW_START = HALF - 128
DEG_HALF = NN_PAD // 2
DEG_STRIP = DEG_HALF // NS

_mesh = plsc.VectorSubcoreMesh(
    core_axis_name="c", subcore_axis_name="s", num_cores=NC, num_subcores=NS)

_f32 = jnp.float32
_i32 = jnp.int32


def _zero_fill(ref, rows):
    z = jnp.zeros((16,), _f32)

    def body(r, _):
        for q in range(4):
            ref[r, pl.ds(q * 16, 16)] = z
        return 0

    lax.fori_loop(0, rows, body, 0)


L_ROWS = 224
L_CAP = L_ROWS * 128
NREG = NW * 2


@functools.partial(
    pl.kernel,
    out_type=(
        jax.ShapeDtypeStruct((NREG * L_CAP,), _i32),
        jax.ShapeDtypeStruct((NREG * L_CAP,), _i32),
        jax.ShapeDtypeStruct((NREG * 16,), _i32),
    ),
    mesh=_mesh,
    compiler_params=pltpu.CompilerParams(use_tc_tiling_on_sc=False, needs_layout_passes=False),
    scratch_types=[
        pltpu.VMEM((8, 128), _i32),
        pltpu.VMEM((8, 128), _i32),
        pltpu.VMEM((L_CAP,), _i32),
        pltpu.VMEM((L_CAP,), _i32),
        pltpu.VMEM((L_CAP,), _i32),
        pltpu.VMEM((L_CAP,), _i32),
        pltpu.VMEM((16,), _i32),
    ],
)
def _partition_edges(src_h, dst_h, lsrc, ldst, cnts,
                     ins, ind, s0, d0, s1, d1, cb):
    c = lax.axis_index("c")
    s = lax.axis_index("s")
    wid = c * NS + s
    lane = lax.iota(_i32, 16)
    nblk = jnp.where(wid < 16, 25, 24)
    b0_ = jnp.where(wid < 16, 25 * wid, 400 + 24 * (wid - 16))

    def blk(k, carry):
        p0, p1 = carry
        r = (b0_ + k) * 8
        pltpu.sync_copy(src_h.at[pl.ds(r, 8)], ins)
        pltpu.sync_copy(dst_h.at[pl.ds(r, 8)], ind)
        for i in range(8):
            for q in range(8):
                sl = (i, pl.ds(q * 16, 16))
                sv = ins[sl]
                dv = ind[sl]
                m0 = dv < HALF
                m1 = jnp.logical_not(m0)
                plsc.store_compressed(s0.at[pl.ds(p0, 16)], sv, mask=m0)
                plsc.store_compressed(d0.at[pl.ds(p0, 16)], dv, mask=m0)
                plsc.store_compressed(s1.at[pl.ds(p1, 16)], sv, mask=m1)
                plsc.store_compressed(
                    d1.at[pl.ds(p1, 16)], dv - HALF, mask=m1)
                pc = jnp.sum(jnp.where(m0, 1, 0))
                p0 = p0 + pc
                p1 = p1 + (16 - pc)
        return p0, p1

    p0, p1 = lax.fori_loop(0, nblk, blk, (jnp.int32(0), jnp.int32(0)))

    dummy_s = jnp.full((16,), NN, _i32)
    dummy_d = HALF + lane
    for t in range(8):
        s0[pl.ds(p0 + t * 16, 16)] = dummy_s
        d0[pl.ds(p0 + t * 16, 16)] = dummy_d
        s1[pl.ds(p1 + t * 16, 16)] = dummy_s
        d1[pl.ds(p1 + t * 16, 16)] = dummy_d
    nr0 = lax.shift_right_logical(p0 + 127, 7)
    nr1 = lax.shift_right_logical(p1 + 127, 7)

    rb0 = (wid * 2 + 0) * L_CAP
    rb1 = (wid * 2 + 1) * L_CAP
    pltpu.sync_copy(s0, lsrc.at[pl.ds(rb0, L_CAP)])
    pltpu.sync_copy(d0, ldst.at[pl.ds(rb0, L_CAP)])
    pltpu.sync_copy(s1, lsrc.at[pl.ds(rb1, L_CAP)])
    pltpu.sync_copy(d1, ldst.at[pl.ds(rb1, L_CAP)])
    cb[pl.ds(0, 16)] = jnp.where(lane >= 0, nr0, 0)
    pltpu.sync_copy(cb, cnts.at[pl.ds((wid * 2 + 0) * 16, 16)])
    cb[pl.ds(0, 16)] = jnp.where(lane >= 0, nr1, 0)
    pltpu.sync_copy(cb, cnts.at[pl.ds((wid * 2 + 1) * 16, 16)])


@functools.partial(
    pl.kernel,
    out_type=(
        jax.ShapeDtypeStruct((NN_PAD,), _f32),
        jax.ShapeDtypeStruct((NN_PAD,), _f32),
    ),
    mesh=_mesh,
    compiler_params=pltpu.CompilerParams(use_tc_tiling_on_sc=False, needs_layout_passes=False),
    scratch_types=[
        pltpu.VMEM_SHARED((NN_PAD,), _f32),
        pltpu.VMEM_SHARED((NN_PAD,), _f32),
        pltpu.VMEM((DEG_STRIP,), _f32),
        pltpu.VMEM((8, 128), _i32),
        pltpu.VMEM((128,), _f32),
    ],
)
def _hist(srcp, dstp, degs_h, degd_h, deg_s, deg_d, zb, ib, ones):
    c = lax.axis_index("c")
    s = lax.axis_index("s")

    def zfill(i, _):
        zb[pl.ds(i * 16, 16)] = jnp.zeros((16,), _f32)
        return 0

    lax.fori_loop(0, DEG_STRIP // 16, zfill, 0)
    for q in range(8):
        ones[pl.ds(q * 16, 16)] = jnp.ones((16,), _f32)
    for h in range(2):
        strip = (h * NS + s) * DEG_STRIP
        pltpu.sync_copy(zb, deg_s.at[pl.ds(strip, DEG_STRIP)])
        pltpu.sync_copy(zb, deg_d.at[pl.ds(strip, DEG_STRIP)])
    plsc.subcore_barrier()

    def hist(g, _):
        r = s * EPT_ROWS + g * 8
        pltpu.sync_copy(srcp.at[pl.ds(r, 8)], ib)
        for j in range(8):
            pltpu.sync_copy(ones, deg_s.at[ib.at[j]], add=True)
        pltpu.sync_copy(dstp.at[pl.ds(r, 8)], ib)
        for j in range(8):
            pltpu.sync_copy(ones, deg_d.at[ib.at[j]], add=True)
        return 0

    lax.fori_loop(0, N_EDGE_BLK, hist, 0)
    plsc.subcore_barrier()

    base = c * DEG_HALF + s * DEG_STRIP
    pltpu.sync_copy(deg_s.at[pl.ds(base, DEG_STRIP)], zb)
    pltpu.sync_copy(zb, degs_h.at[pl.ds(base, DEG_STRIP)])
    pltpu.sync_copy(deg_d.at[pl.ds(base, DEG_STRIP)], zb)
    pltpu.sync_copy(zb, degd_h.at[pl.ds(base, DEG_STRIP)])


_TC_BLK = 1024
_TC_GRID = NN_PAD // _TC_BLK


def _prep_tc(deg_s, deg_d, t0):
    def body(ds_ref, dd_ref, t0_ref, win_ref, wprod_ref, t0p_ref):
        iso = lax.rsqrt(jnp.maximum(ds_ref[...], 1.0))
        isi = lax.rsqrt(jnp.maximum(dd_ref[...], 1.0))
        win_ref[...] = isi
        wprod_ref[...] = isi * iso
        t0p_ref[...] = t0_ref[...] * iso

    return pl.pallas_call(
        body,
        grid=(_TC_GRID,),
        in_specs=[
            pl.BlockSpec((_TC_BLK, 1), lambda i: (i, 0)),
            pl.BlockSpec((_TC_BLK, 1), lambda i: (i, 0)),
            pl.BlockSpec((_TC_BLK, D), lambda i: (i, 0)),
        ],
        out_specs=[
            pl.BlockSpec((_TC_BLK, 1), lambda i: (i, 0)),
            pl.BlockSpec((_TC_BLK, 1), lambda i: (i, 0)),
            pl.BlockSpec((_TC_BLK, D), lambda i: (i, 0)),
        ],
        out_shape=[
            jax.ShapeDtypeStruct((NN_PAD, 1), _f32),
            jax.ShapeDtypeStruct((NN_PAD, 1), _f32),
            jax.ShapeDtypeStruct((NN_PAD, D), _f32),
        ],
    )(deg_s, deg_d, t0)


def _scale_tc(s_tab, win, wprod):
    def body(s_ref, win_ref, wprod_ref, t_ref, tp_ref):
        sv = s_ref[...]
        t_ref[...] = sv * win_ref[...]
        tp_ref[...] = sv * wprod_ref[...]

    return pl.pallas_call(
        body,
        grid=(_TC_GRID,),
        in_specs=[
            pl.BlockSpec((_TC_BLK, D), lambda i: (i, 0)),
            pl.BlockSpec((_TC_BLK, 1), lambda i: (i, 0)),
            pl.BlockSpec((_TC_BLK, 1), lambda i: (i, 0)),
        ],
        out_specs=[
            pl.BlockSpec((_TC_BLK, D), lambda i: (i, 0)),
            pl.BlockSpec((_TC_BLK, D), lambda i: (i, 0)),
        ],
        out_shape=[
            jax.ShapeDtypeStruct((NN_PAD, D), _f32),
            jax.ShapeDtypeStruct((NN_PAD, D), _f32),
        ],
    )(s_tab, win, wprod)


@functools.partial(
    pl.kernel,
    out_type=jax.ShapeDtypeStruct((NN_PAD, D), _f32),
    mesh=_mesh,
    compiler_params=pltpu.CompilerParams(use_tc_tiling_on_sc=False, needs_layout_passes=False),
    scratch_types=[
        pltpu.VMEM_SHARED((ACC_ROWS, D), _f32),
        pltpu.VMEM((32, 128), _i32),
        pltpu.VMEM((32, 128), _i32),
        pltpu.VMEM((16,), _i32),
        pltpu.VMEM((256, D), _f32),
        pltpu.SemaphoreType.DMA,
    ],
)
def _layer(tp_in, lsrc, ldst, cnts, s_out, acc, sb, db, cv, rows, sem):
    c = lax.axis_index("c")
    s = lax.axis_index("s")
    lane = lax.iota(_i32, 16)

    _zero_fill(rows, 128)
    zbase = s * (ACC_ROWS // NS)

    def zloop(k, _):
        start = zbase + jnp.minimum(k * 128, ACC_ROWS // NS - 128)
        pltpu.sync_copy(rows.at[pl.ds(0, 128)], acc.at[pl.ds(start, 128)])
        return 0

    lax.fori_loop(0, 13, zloop, 0)
    plsc.subcore_barrier()

    def dbuf(r):
        return rows.at[pl.ds(lax.rem(r, 2) * 128, 128)]

    for reg in range(2):
        regid = (2 * s + reg) * 2 + c
        pltpu.sync_copy(cnts.at[pl.ds(regid * 16, 16)], cv)
        nr = jnp.sum(jnp.where(lane == 0, cv[pl.ds(0, 16)], 0))
        rowbase = regid * L_ROWS

        def sub(k, _):
            @pl.when(k * 32 < nr)
            def _():
                pltpu.sync_copy(lsrc.at[pl.ds(rowbase + k * 32, 32)], sb)
                pltpu.sync_copy(ldst.at[pl.ds(rowbase + k * 32, 32)], db)
                nrr = jnp.minimum(nr - k * 32, 32)
                pltpu.async_copy(tp_in.at[sb.at[0]], dbuf(0), sem)

                def inner(r, _):
                    pltpu.make_async_copy(
                        tp_in.at[pl.ds(0, 128)], dbuf(r), sem).wait()

                    @pl.when(r + 1 < nrr)
                    def _():
                        pltpu.async_copy(
                            tp_in.at[sb.at[r + 1]], dbuf(r + 1), sem)

                    pltpu.sync_copy(dbuf(r), acc.at[db.at[r]], add=True)
                    return 0

                lax.fori_loop(0, nrr, inner, 0)
            return 0

        lax.fori_loop(0, L_ROWS // 32, sub, 0)
    plsc.subcore_barrier()

    def rows_out(j, _):
        g = j * NS + s

        @pl.when(g < ROW_BLOCKS)
        def _():
            lstart = jnp.minimum(g * 128, LAST_ROW_START)
            gstart = c * HALF + lstart
            pltpu.sync_copy(acc.at[pl.ds(lstart, 128)], rows.at[pl.ds(0, 128)])
            pltpu.sync_copy(rows.at[pl.ds(0, 128)], s_out.at[pl.ds(gstart, 128)])
        return 0

    lax.fori_loop(0, 13, rows_out, 0)


@functools.partial(
    pl.kernel,
    out_type=jax.ShapeDtypeStruct((BATCH,), _f32),
    mesh=_mesh,
    compiler_params=pltpu.CompilerParams(use_tc_tiling_on_sc=False, needs_layout_passes=False),
    scratch_types=[
        pltpu.VMEM((128,), _i32),
        pltpu.VMEM((128,), _i32),
        pltpu.VMEM((128, D), _f32), pltpu.VMEM((128, D), _f32),
        pltpu.VMEM((128, D), _f32), pltpu.VMEM((128, D), _f32),
        pltpu.VMEM((128, D), _f32), pltpu.VMEM((128, D), _f32),
        pltpu.VMEM((128, D), _f32), pltpu.VMEM((128, D), _f32),
        pltpu.VMEM((128,), _f32),
        pltpu.SemaphoreType.DMA,
    ],
)
def _final(users_h, items_h, t0, t1, t2, t3, gamma_h,
           ub, vb, u0, u1, u2, u3, i0, i1, i2, i3, gb, sem):
    c = lax.axis_index("c")
    s = lax.axis_index("s")
    wid = c * NS + s

    def block(t, _):
        sb = wid * 4 + t
        pltpu.sync_copy(users_h.at[pl.ds(sb * 128, 128)], ub)
        pltpu.sync_copy(items_h.at[pl.ds(sb * 128, 128)], vb)
        for q in range(8):
            sl = pl.ds(q * 16, 16)
            vb[sl] = vb[sl] + NU
        cps = [
            pltpu.async_copy(t0.at[ub], u0, sem),
            pltpu.async_copy(t1.at[ub], u1, sem),
            pltpu.async_copy(t2.at[ub], u2, sem),
            pltpu.async_copy(t3.at[ub], u3, sem),
            pltpu.async_copy(t0.at[vb], i0, sem),
            pltpu.async_copy(t1.at[vb], i1, sem),
            pltpu.async_copy(t2.at[vb], i2, sem),
            pltpu.async_copy(t3.at[vb], i3, sem),
        ]
        for cp in cps:
            cp.wait()

        lane = lax.iota(_i32, 16)

        def dot(rb, _):
            res = jnp.zeros((16,), _f32)
            for rr in range(16):
                r = rb * 16 + rr
                acc = jnp.zeros((16,), _f32)
                for q in range(4):
                    sl = (r, pl.ds(q * 16, 16))
                    fu = (u0[sl] + u1[sl]) + (u2[sl] + u3[sl])
                    fi = (i0[sl] + i1[sl]) + (i2[sl] + i3[sl])
                    acc = acc + fu * fi
                res = res + jnp.where(lane == rr, jnp.sum(acc), 0.0)
            gb[pl.ds(rb * 16, 16)] = res * 0.0625
            return 0

        lax.fori_loop(0, 8, dot, 0)
        pltpu.sync_copy(gb, gamma_h.at[pl.ds(sb * 128, 128)])
        return 0

    lax.fori_loop(0, 4, block, 0)


def kernel(users, items, edge_index, user_emb, item_emb):
    srcp = jnp.pad(edge_index[0], (0, E_PAD - E),
                   constant_values=NN).reshape(ER_PAD, 128)
    dstp = jnp.pad(edge_index[1], (0, E_PAD - E),
                   constant_values=NN).reshape(ER_PAD, 128)
    t0 = jnp.concatenate([user_emb, item_emb], axis=0)
    t0 = jnp.pad(t0, ((0, NN_PAD - NN), (0, 0)))

    lsrc, ldst, cnts = _partition_edges(srcp, dstp)
    lsrc2 = lsrc.reshape(NREG * L_ROWS, 128)
    ldst2 = ldst.reshape(NREG * L_ROWS, 128)
    deg_s, deg_d = _hist(srcp, dstp)
    win, wprod, tp = _prep_tc(
        deg_s.reshape(NN_PAD, 1), deg_d.reshape(NN_PAD, 1), t0)
    s1 = _layer(tp, lsrc2, ldst2, cnts)
    t1, tp = _scale_tc(s1, win, wprod)
    s2 = _layer(tp, lsrc2, ldst2, cnts)
    t2, tp = _scale_tc(s2, win, wprod)
    s3 = _layer(tp, lsrc2, ldst2, cnts)
    t3, _ = _scale_tc(s3, win, wprod)

    return _final(users, items, t0, t1, t2, t3)


# speedup vs baseline: 16.7474x; 1.2931x over previous
import functools

import jax
import jax.numpy as jnp
from jax import lax
from jax.experimental import pallas as pl
from jax.experimental.pallas import tpu as pltpu
from jax.experimental.pallas import tpu_sc as plsc

NU = 25000
NN = 50000
D = 64
E = 800000
BATCH = 16384

NC, NS = 2, 16
NW = NC * NS
HALF = 25000
ACC_ROWS = 25088
NN_PAD = 50176
ER = E // 128
ER_PAD = 6272
E_PAD = ER_PAD * 128
EPT_ROWS = ER_PAD // NS
N_EDGE_BLK = EPT_ROWS // 8
ROW_BLOCKS = 196
LAST_ROW_START = HALF - 128
DEG_HALF = NN_PAD // 2
DEG_STRIP = DEG_HALF // NS

_mesh = plsc.VectorSubcoreMesh(
    core_axis_name="c", subcore_axis_name="s", num_cores=NC, num_subcores=NS)

_f32 = jnp.float32
_i32 = jnp.int32


def _zero_fill(ref, rows):
    z = jnp.zeros((16,), _f32)

    def body(r, _):
        for q in range(4):
            ref[r, pl.ds(q * 16, 16)] = z
        return 0

    lax.fori_loop(0, rows, body, 0)


L_ROWS = 224
L_CAP = L_ROWS * 128
NREG = NW * 2


@functools.partial(
    pl.kernel,
    out_type=(
        jax.ShapeDtypeStruct((NREG * L_CAP,), _i32),
        jax.ShapeDtypeStruct((NREG * L_CAP,), _i32),
        jax.ShapeDtypeStruct((NREG * 16,), _i32),
    ),
    mesh=_mesh,
    compiler_params=pltpu.CompilerParams(use_tc_tiling_on_sc=False, needs_layout_passes=False),
    scratch_types=[
        pltpu.VMEM((8, 128), _i32),
        pltpu.VMEM((8, 128), _i32),
        pltpu.VMEM((L_CAP,), _i32),
        pltpu.VMEM((L_CAP,), _i32),
        pltpu.VMEM((L_CAP,), _i32),
        pltpu.VMEM((L_CAP,), _i32),
        pltpu.VMEM((16,), _i32),
    ],
)
def _partition_edges(src_h, dst_h, lsrc, ldst, cnts,
                     ins, ind, s0, d0, s1, d1, cb):
    c = lax.axis_index("c")
    s = lax.axis_index("s")
    wid = c * NS + s
    lane = lax.iota(_i32, 16)
    nblk = jnp.where(wid < 16, 25, 24)
    b0_ = jnp.where(wid < 16, 25 * wid, 400 + 24 * (wid - 16))

    def blk(k, carry):
        p0, p1 = carry
        r = (b0_ + k) * 8
        pltpu.sync_copy(src_h.at[pl.ds(r, 8)], ins)
        pltpu.sync_copy(dst_h.at[pl.ds(r, 8)], ind)
        for i in range(8):
            for q in range(8):
                sl = (i, pl.ds(q * 16, 16))
                sv = ins[sl]
                dv = ind[sl]
                m0 = dv < HALF
                m1 = jnp.logical_not(m0)
                plsc.store_compressed(s0.at[pl.ds(p0, 16)], sv, mask=m0)
                plsc.store_compressed(d0.at[pl.ds(p0, 16)], dv, mask=m0)
                plsc.store_compressed(s1.at[pl.ds(p1, 16)], sv, mask=m1)
                plsc.store_compressed(
                    d1.at[pl.ds(p1, 16)], dv - HALF, mask=m1)
                pc = jnp.sum(jnp.where(m0, 1, 0))
                p0 = p0 + pc
                p1 = p1 + (16 - pc)
        return p0, p1

    p0, p1 = lax.fori_loop(0, nblk, blk, (jnp.int32(0), jnp.int32(0)))

    dummy_s = jnp.full((16,), NN, _i32)
    dummy_d = HALF + lane
    for t in range(8):
        s0[pl.ds(p0 + t * 16, 16)] = dummy_s
        d0[pl.ds(p0 + t * 16, 16)] = dummy_d
        s1[pl.ds(p1 + t * 16, 16)] = dummy_s
        d1[pl.ds(p1 + t * 16, 16)] = dummy_d
    nr0 = lax.shift_right_logical(p0 + 127, 7)
    nr1 = lax.shift_right_logical(p1 + 127, 7)

    rb0 = (wid * 2 + 0) * L_CAP
    rb1 = (wid * 2 + 1) * L_CAP
    pltpu.sync_copy(s0, lsrc.at[pl.ds(rb0, L_CAP)])
    pltpu.sync_copy(d0, ldst.at[pl.ds(rb0, L_CAP)])
    pltpu.sync_copy(s1, lsrc.at[pl.ds(rb1, L_CAP)])
    pltpu.sync_copy(d1, ldst.at[pl.ds(rb1, L_CAP)])
    cb[pl.ds(0, 16)] = jnp.where(lane >= 0, nr0, 0)
    pltpu.sync_copy(cb, cnts.at[pl.ds((wid * 2 + 0) * 16, 16)])
    cb[pl.ds(0, 16)] = jnp.where(lane >= 0, nr1, 0)
    pltpu.sync_copy(cb, cnts.at[pl.ds((wid * 2 + 1) * 16, 16)])


@functools.partial(
    pl.kernel,
    out_type=(
        jax.ShapeDtypeStruct((2 * NN_PAD,), _f32),
        jax.ShapeDtypeStruct((2 * NN_PAD,), _f32),
    ),
    mesh=_mesh,
    compiler_params=pltpu.CompilerParams(use_tc_tiling_on_sc=False, needs_layout_passes=False),
    scratch_types=[
        pltpu.VMEM_SHARED((NN_PAD,), _f32),
        pltpu.VMEM_SHARED((NN_PAD,), _f32),
        pltpu.VMEM((DEG_STRIP,), _f32),
        pltpu.VMEM((8, 128), _i32),
        pltpu.VMEM((128,), _f32),
    ],
)
def _hist(srcp, dstp, degs_h, degd_h, deg_s, deg_d, zb, ib, ones):
    c = lax.axis_index("c")
    s = lax.axis_index("s")

    def zfill(i, _):
        zb[pl.ds(i * 16, 16)] = jnp.zeros((16,), _f32)
        return 0

    lax.fori_loop(0, DEG_STRIP // 16, zfill, 0)
    for q in range(8):
        ones[pl.ds(q * 16, 16)] = jnp.ones((16,), _f32)
    for h in range(2):
        strip = (h * NS + s) * DEG_STRIP
        pltpu.sync_copy(zb, deg_s.at[pl.ds(strip, DEG_STRIP)])
        pltpu.sync_copy(zb, deg_d.at[pl.ds(strip, DEG_STRIP)])
    plsc.subcore_barrier()

    wid = c * NS + s

    def hist(k, _):
        b = k * NW + wid

        @pl.when(b < ER_PAD // 8)
        def _():
            r = b * 8
            pltpu.sync_copy(srcp.at[pl.ds(r, 8)], ib)
            for j in range(8):
                pltpu.sync_copy(ones, deg_s.at[ib.at[j]], add=True)
            pltpu.sync_copy(dstp.at[pl.ds(r, 8)], ib)
            for j in range(8):
                pltpu.sync_copy(ones, deg_d.at[ib.at[j]], add=True)
        return 0

    lax.fori_loop(0, (ER_PAD // 8 + NW - 1) // NW, hist, 0)
    plsc.subcore_barrier()

    for h in range(2):
        off = s * (2 * DEG_STRIP) + h * DEG_STRIP
        base = c * NN_PAD + off
        pltpu.sync_copy(deg_s.at[pl.ds(off, DEG_STRIP)], zb)
        pltpu.sync_copy(zb, degs_h.at[pl.ds(base, DEG_STRIP)])
        pltpu.sync_copy(deg_d.at[pl.ds(off, DEG_STRIP)], zb)
        pltpu.sync_copy(zb, degd_h.at[pl.ds(base, DEG_STRIP)])


_TC_BLK = 1024
_TC_GRID = NN_PAD // _TC_BLK


def _prep_tc(ds0, ds1, dd0, dd1, t0):
    def body(ds0_ref, ds1_ref, dd0_ref, dd1_ref, t0_ref,
             win_ref, wprod_ref, t0p_ref):
        iso = lax.rsqrt(jnp.maximum(ds0_ref[...] + ds1_ref[...], 1.0))
        isi = lax.rsqrt(jnp.maximum(dd0_ref[...] + dd1_ref[...], 1.0))
        win_ref[...] = isi
        wprod_ref[...] = isi * iso
        t0p_ref[...] = t0_ref[...] * iso

    vec = pl.BlockSpec((_TC_BLK, 1), lambda i: (i, 0))
    mat = pl.BlockSpec((_TC_BLK, D), lambda i: (i, 0))
    return pl.pallas_call(
        body,
        grid=(_TC_GRID,),
        in_specs=[vec, vec, vec, vec, mat],
        out_specs=[vec, vec, mat],
        out_shape=[
            jax.ShapeDtypeStruct((NN_PAD, 1), _f32),
            jax.ShapeDtypeStruct((NN_PAD, 1), _f32),
            jax.ShapeDtypeStruct((NN_PAD, D), _f32),
        ],
    )(ds0, ds1, dd0, dd1, t0)


@functools.partial(
    pl.kernel,
    out_type=(
        jax.ShapeDtypeStruct((NN_PAD, D), _f32),
        jax.ShapeDtypeStruct((NN_PAD, D), _f32),
    ),
    mesh=_mesh,
    compiler_params=pltpu.CompilerParams(use_tc_tiling_on_sc=False, needs_layout_passes=False),
    scratch_types=[
        pltpu.VMEM_SHARED((ACC_ROWS, D), _f32),
        pltpu.VMEM((16, 128), _i32),
        pltpu.VMEM((16, 128), _i32),
        pltpu.VMEM((16,), _i32),
        pltpu.VMEM((128,), _f32),
        pltpu.VMEM((128,), _f32),
        pltpu.VMEM((384, D), _f32),
        pltpu.SemaphoreType.DMA,
        pltpu.SemaphoreType.DMA,
    ],
)
def _layer(tp_in, lsrc, ldst, cnts, win_h, wprod_h, t_out, tp_out,
           acc, sb, db, cv, wv, wpv, rows, sem, sem2):
    c = lax.axis_index("c")
    s = lax.axis_index("s")
    lane = lax.iota(_i32, 16)

    _zero_fill(rows, 128)
    zbase = s * (ACC_ROWS // NS)

    def zloop(k, _):
        start = zbase + jnp.minimum(k * 128, ACC_ROWS // NS - 128)
        pltpu.sync_copy(rows.at[pl.ds(0, 128)], acc.at[pl.ds(start, 128)])
        return 0

    lax.fori_loop(0, 13, zloop, 0)
    plsc.subcore_barrier()

    def dbuf(r):
        return rows.at[pl.ds(lax.rem(r, 3) * 128, 128)]

    def drain(sem_):
        pltpu.make_async_copy(
            tp_in.at[pl.ds(0, 128)], rows.at[pl.ds(0, 128)], sem_).wait()

    for reg in range(2):
        regid = (2 * s + reg) * 2 + c
        pltpu.sync_copy(cnts.at[pl.ds(regid * 16, 16)], cv)
        nr = jnp.sum(jnp.where(lane == 0, cv[pl.ds(0, 16)], 0))
        rowbase = regid * L_ROWS

        def sub(k, _):
            @pl.when(k * 16 < nr)
            def _():
                pltpu.sync_copy(lsrc.at[pl.ds(rowbase + k * 16, 16)], sb)
                pltpu.sync_copy(ldst.at[pl.ds(rowbase + k * 16, 16)], db)
                nrr = jnp.minimum(nr - k * 16, 16)
                pltpu.async_copy(tp_in.at[sb.at[0]], dbuf(0), sem)

                @pl.when(nrr > 1)
                def _():
                    pltpu.async_copy(tp_in.at[sb.at[1]], dbuf(1), sem)

                def inner(r, _):
                    drain(sem)
                    pltpu.async_copy(
                        dbuf(r), acc.at[db.at[r]], sem2, add=True)

                    @pl.when(r >= 1)
                    def _():
                        drain(sem2)

                    @pl.when(r + 2 < nrr)
                    def _():
                        pltpu.async_copy(
                            tp_in.at[sb.at[r + 2]], dbuf(r + 2), sem)
                    return 0

                lax.fori_loop(0, nrr, inner, 0)
                drain(sem2)
            return 0

        lax.fori_loop(0, L_ROWS // 16, sub, 0)
    plsc.subcore_barrier()

    def rows_out(j, _):
        g = j * NS + s

        @pl.when(g < ROW_BLOCKS)
        def _():
            lstart = jnp.minimum(g * 128, LAST_ROW_START)
            gstart = c * HALF + lstart
            pltpu.sync_copy(acc.at[pl.ds(lstart, 128)], rows.at[pl.ds(0, 128)])
            pltpu.sync_copy(win_h.at[pl.ds(gstart, 128)], wv)
            pltpu.sync_copy(wprod_h.at[pl.ds(gstart, 128)], wpv)

            def scale(r, _):
                idx = jnp.full((16,), r, _i32)
                w = plsc.load_gather(wv, [idx])
                wp = plsc.load_gather(wpv, [idx])
                for q in range(4):
                    sl = pl.ds(q * 16, 16)
                    sv = rows[r, sl]
                    rows[r + 128, sl] = sv * w
                    rows[r + 256, sl] = sv * wp
                return 0

            lax.fori_loop(0, 128, scale, 0)
            pltpu.sync_copy(rows.at[pl.ds(128, 128)], t_out.at[pl.ds(gstart, 128)])
            pltpu.sync_copy(rows.at[pl.ds(256, 128)], tp_out.at[pl.ds(gstart, 128)])
        return 0

    lax.fori_loop(0, 13, rows_out, 0)


@functools.partial(
    pl.kernel,
    out_type=jax.ShapeDtypeStruct((BATCH,), _f32),
    mesh=_mesh,
    compiler_params=pltpu.CompilerParams(use_tc_tiling_on_sc=False, needs_layout_passes=False),
    scratch_types=[
        pltpu.VMEM((128,), _i32),
        pltpu.VMEM((128,), _i32),
        pltpu.VMEM((128, D), _f32), pltpu.VMEM((128, D), _f32),
        pltpu.VMEM((128, D), _f32), pltpu.VMEM((128, D), _f32),
        pltpu.VMEM((128, D), _f32), pltpu.VMEM((128, D), _f32),
        pltpu.VMEM((128, D), _f32), pltpu.VMEM((128, D), _f32),
        pltpu.VMEM((128,), _f32),
        pltpu.SemaphoreType.DMA,
    ],
)
def _final(users_h, items_h, t0, t1, t2, t3, gamma_h,
           ub, vb, u0, u1, u2, u3, i0, i1, i2, i3, gb, sem):
    c = lax.axis_index("c")
    s = lax.axis_index("s")
    wid = c * NS + s

    def block(t, _):
        sb = wid * 4 + t
        pltpu.sync_copy(users_h.at[pl.ds(sb * 128, 128)], ub)
        pltpu.sync_copy(items_h.at[pl.ds(sb * 128, 128)], vb)
        for q in range(8):
            sl = pl.ds(q * 16, 16)
            vb[sl] = vb[sl] + NU
        cps = [
            pltpu.async_copy(t0.at[ub], u0, sem),
            pltpu.async_copy(t1.at[ub], u1, sem),
            pltpu.async_copy(t2.at[ub], u2, sem),
            pltpu.async_copy(t3.at[ub], u3, sem),
            pltpu.async_copy(t0.at[vb], i0, sem),
            pltpu.async_copy(t1.at[vb], i1, sem),
            pltpu.async_copy(t2.at[vb], i2, sem),
            pltpu.async_copy(t3.at[vb], i3, sem),
        ]
        for cp in cps:
            cp.wait()

        lane = lax.iota(_i32, 16)

        def dot(rb, _):
            res = jnp.zeros((16,), _f32)
            for rr in range(16):
                r = rb * 16 + rr
                acc = jnp.zeros((16,), _f32)
                for q in range(4):
                    sl = (r, pl.ds(q * 16, 16))
                    fu = (u0[sl] + u1[sl]) + (u2[sl] + u3[sl])
                    fi = (i0[sl] + i1[sl]) + (i2[sl] + i3[sl])
                    acc = acc + fu * fi
                res = res + jnp.where(lane == rr, jnp.sum(acc), 0.0)
            gb[pl.ds(rb * 16, 16)] = res * 0.0625
            return 0

        lax.fori_loop(0, 8, dot, 0)
        pltpu.sync_copy(gb, gamma_h.at[pl.ds(sb * 128, 128)])
        return 0

    lax.fori_loop(0, 4, block, 0)


def kernel(users, items, edge_index, user_emb, item_emb):
    srcp = jnp.pad(edge_index[0], (0, E_PAD - E),
                   constant_values=NN).reshape(ER_PAD, 128)
    dstp = jnp.pad(edge_index[1], (0, E_PAD - E),
                   constant_values=NN).reshape(ER_PAD, 128)
    t0 = jnp.concatenate([user_emb, item_emb], axis=0)
    t0 = jnp.pad(t0, ((0, NN_PAD - NN), (0, 0)))

    lsrc, ldst, cnts = _partition_edges(srcp, dstp)
    lsrc2 = lsrc.reshape(NREG * L_ROWS, 128)
    ldst2 = ldst.reshape(NREG * L_ROWS, 128)
    deg_s, deg_d = _hist(srcp, dstp)
    ds2 = deg_s.reshape(2, NN_PAD, 1)
    dd2 = deg_d.reshape(2, NN_PAD, 1)
    win, wprod, tp = _prep_tc(ds2[0], ds2[1], dd2[0], dd2[1], t0)
    win1 = win.reshape(NN_PAD)
    wprod1 = wprod.reshape(NN_PAD)
    t1, tp = _layer(tp, lsrc2, ldst2, cnts, win1, wprod1)
    t2, tp = _layer(tp, lsrc2, ldst2, cnts, win1, wprod1)
    t3, _ = _layer(tp, lsrc2, ldst2, cnts, win1, wprod1)

    return _final(users, items, t0, t1, t2, t3)


# speedup vs baseline: 17.1445x; 1.0237x over previous
import functools

import jax
import jax.numpy as jnp
from jax import lax
from jax.experimental import pallas as pl
from jax.experimental.pallas import tpu as pltpu
from jax.experimental.pallas import tpu_sc as plsc

NU = 25000
NN = 50000
D = 64
E = 800000
BATCH = 16384

NC, NS = 2, 16
NW = NC * NS
HALF = 25000
ACC_ROWS = 25088
NN_PAD = 50176
ER = E // 128
ER_PAD = 6272
E_PAD = ER_PAD * 128
EPT_ROWS = ER_PAD // NS
N_EDGE_BLK = EPT_ROWS // 8
ROW_BLOCKS = 196
LAST_ROW_START = HALF - 128
DEG_HALF = NN_PAD // 2
DEG_STRIP = DEG_HALF // NS

_mesh = plsc.VectorSubcoreMesh(
    core_axis_name="c", subcore_axis_name="s", num_cores=NC, num_subcores=NS)

_f32 = jnp.float32
_i32 = jnp.int32


def _zero_fill(ref, rows):
    z = jnp.zeros((16,), _f32)

    def body(r, _):
        for q in range(4):
            ref[r, pl.ds(q * 16, 16)] = z
        return 0

    lax.fori_loop(0, rows, body, 0)


L_ROWS = 224
L_CAP = L_ROWS * 128
NREG = NW * 2


@functools.partial(
    pl.kernel,
    out_type=(
        jax.ShapeDtypeStruct((NREG * L_CAP,), _i32),
        jax.ShapeDtypeStruct((NREG * L_CAP,), _i32),
        jax.ShapeDtypeStruct((NREG * 16,), _i32),
        jax.ShapeDtypeStruct((2 * NN_PAD,), _f32),
        jax.ShapeDtypeStruct((2 * NN_PAD,), _f32),
    ),
    mesh=_mesh,
    compiler_params=pltpu.CompilerParams(use_tc_tiling_on_sc=False, needs_layout_passes=False),
    scratch_types=[
        pltpu.VMEM_SHARED((NN_PAD,), _f32),
        pltpu.VMEM_SHARED((NN_PAD,), _f32),
        pltpu.VMEM((8, 128), _i32),
        pltpu.VMEM((8, 128), _i32),
        pltpu.VMEM((L_CAP,), _i32),
        pltpu.VMEM((L_CAP,), _i32),
        pltpu.VMEM((L_CAP,), _i32),
        pltpu.VMEM((L_CAP,), _i32),
        pltpu.VMEM((DEG_STRIP,), _f32),
        pltpu.VMEM((128,), _f32),
        pltpu.VMEM((16,), _i32),
        pltpu.SemaphoreType.DMA,
    ],
)
def _partition_edges(src_h, dst_h, lsrc, ldst, cnts, degs_h, degd_h,
                     deg_s, deg_d, ins, ind, s0, d0, s1, d1, zb, ones, cb,
                     hsem):
    c = lax.axis_index("c")
    s = lax.axis_index("s")
    wid = c * NS + s
    lane = lax.iota(_i32, 16)

    def zfill(i, _):
        zb[pl.ds(i * 16, 16)] = jnp.zeros((16,), _f32)
        return 0

    lax.fori_loop(0, DEG_STRIP // 16, zfill, 0)
    for q in range(8):
        ones[pl.ds(q * 16, 16)] = jnp.ones((16,), _f32)
    for h in range(2):
        strip = (h * NS + s) * DEG_STRIP
        pltpu.sync_copy(zb, deg_s.at[pl.ds(strip, DEG_STRIP)])
        pltpu.sync_copy(zb, deg_d.at[pl.ds(strip, DEG_STRIP)])
    plsc.subcore_barrier()

    nblk = jnp.where(wid < 16, 25, 24)
    b0_ = jnp.where(wid < 16, 25 * wid, 400 + 24 * (wid - 16))

    def blk(k, carry):
        p0, p1 = carry
        r = (b0_ + k) * 8
        pltpu.sync_copy(src_h.at[pl.ds(r, 8)], ins)
        pltpu.sync_copy(dst_h.at[pl.ds(r, 8)], ind)
        for j in range(8):
            pltpu.async_copy(ones, deg_s.at[ins.at[j]], hsem, add=True)
            pltpu.async_copy(ones, deg_d.at[ind.at[j]], hsem, add=True)
        for i in range(8):
            for q in range(8):
                sl = (i, pl.ds(q * 16, 16))
                sv = ins[sl]
                dv = ind[sl]
                m0 = dv < HALF
                m1 = jnp.logical_not(m0)
                plsc.store_compressed(s0.at[pl.ds(p0, 16)], sv, mask=m0)
                plsc.store_compressed(d0.at[pl.ds(p0, 16)], dv, mask=m0)
                plsc.store_compressed(s1.at[pl.ds(p1, 16)], sv, mask=m1)
                plsc.store_compressed(
                    d1.at[pl.ds(p1, 16)], dv - HALF, mask=m1)
                pc = jnp.sum(jnp.where(m0, 1, 0))
                p0 = p0 + pc
                p1 = p1 + (16 - pc)
        for _ in range(16):
            pltpu.make_async_copy(
                degs_h.at[pl.ds(0, 128)], ones, hsem).wait()
        return p0, p1

    p0, p1 = lax.fori_loop(0, nblk, blk, (jnp.int32(0), jnp.int32(0)))

    dummy_s = jnp.full((16,), NN, _i32)
    dummy_d = HALF + lane
    for t in range(8):
        s0[pl.ds(p0 + t * 16, 16)] = dummy_s
        d0[pl.ds(p0 + t * 16, 16)] = dummy_d
        s1[pl.ds(p1 + t * 16, 16)] = dummy_s
        d1[pl.ds(p1 + t * 16, 16)] = dummy_d
    nr0 = lax.shift_right_logical(p0 + 127, 7)
    nr1 = lax.shift_right_logical(p1 + 127, 7)

    rb0 = (wid * 2 + 0) * L_CAP
    rb1 = (wid * 2 + 1) * L_CAP
    pltpu.sync_copy(s0, lsrc.at[pl.ds(rb0, L_CAP)])
    pltpu.sync_copy(d0, ldst.at[pl.ds(rb0, L_CAP)])
    pltpu.sync_copy(s1, lsrc.at[pl.ds(rb1, L_CAP)])
    pltpu.sync_copy(d1, ldst.at[pl.ds(rb1, L_CAP)])
    cb[pl.ds(0, 16)] = jnp.where(lane >= 0, nr0, 0)
    pltpu.sync_copy(cb, cnts.at[pl.ds((wid * 2 + 0) * 16, 16)])
    cb[pl.ds(0, 16)] = jnp.where(lane >= 0, nr1, 0)
    pltpu.sync_copy(cb, cnts.at[pl.ds((wid * 2 + 1) * 16, 16)])

    plsc.subcore_barrier()
    for h in range(2):
        off = s * (2 * DEG_STRIP) + h * DEG_STRIP
        base = c * NN_PAD + off
        pltpu.sync_copy(deg_s.at[pl.ds(off, DEG_STRIP)], zb)
        pltpu.sync_copy(zb, degs_h.at[pl.ds(base, DEG_STRIP)])
        pltpu.sync_copy(deg_d.at[pl.ds(off, DEG_STRIP)], zb)
        pltpu.sync_copy(zb, degd_h.at[pl.ds(base, DEG_STRIP)])


_TC_BLK = 1024
_TC_GRID = NN_PAD // _TC_BLK


def _prep_tc(ds0, ds1, dd0, dd1, t0):
    def body(ds0_ref, ds1_ref, dd0_ref, dd1_ref, t0_ref,
             win_ref, wprod_ref, t0p_ref):
        iso = lax.rsqrt(jnp.maximum(ds0_ref[...] + ds1_ref[...], 1.0))
        isi = lax.rsqrt(jnp.maximum(dd0_ref[...] + dd1_ref[...], 1.0))
        win_ref[...] = isi
        wprod_ref[...] = isi * iso
        t0p_ref[...] = t0_ref[...] * iso

    vec = pl.BlockSpec((_TC_BLK, 1), lambda i: (i, 0))
    mat = pl.BlockSpec((_TC_BLK, D), lambda i: (i, 0))
    return pl.pallas_call(
        body,
        grid=(_TC_GRID,),
        in_specs=[vec, vec, vec, vec, mat],
        out_specs=[vec, vec, mat],
        out_shape=[
            jax.ShapeDtypeStruct((NN_PAD, 1), _f32),
            jax.ShapeDtypeStruct((NN_PAD, 1), _f32),
            jax.ShapeDtypeStruct((NN_PAD, D), _f32),
        ],
    )(ds0, ds1, dd0, dd1, t0)


@functools.partial(
    pl.kernel,
    out_type=(
        jax.ShapeDtypeStruct((NN_PAD, D), _f32),
        jax.ShapeDtypeStruct((NN_PAD, D), _f32),
    ),
    mesh=_mesh,
    compiler_params=pltpu.CompilerParams(use_tc_tiling_on_sc=False, needs_layout_passes=False),
    scratch_types=[
        pltpu.VMEM_SHARED((ACC_ROWS, D), _f32),
        pltpu.VMEM((2048,), _i32),
        pltpu.VMEM((16, 128), _i32),
        pltpu.VMEM((16,), _i32),
        pltpu.VMEM((128,), _f32),
        pltpu.VMEM((128,), _f32),
        pltpu.VMEM((384, D), _f32),
        pltpu.SemaphoreType.DMA,
        pltpu.SemaphoreType.DMA,
    ],
)
def _layer(tp_in, lsrc1, ldst, cnts, win_h, wprod_h, t_out, tp_out,
           acc, sb, db, cv, wv, wpv, rows, sem, sem2):
    c = lax.axis_index("c")
    s = lax.axis_index("s")
    lane = lax.iota(_i32, 16)

    _zero_fill(rows, 128)
    zbase = s * (ACC_ROWS // NS)

    def zloop(k, _):
        start = zbase + jnp.minimum(k * 128, ACC_ROWS // NS - 128)
        pltpu.sync_copy(rows.at[pl.ds(0, 128)], acc.at[pl.ds(start, 128)])
        return 0

    lax.fori_loop(0, 13, zloop, 0)
    plsc.subcore_barrier()

    def dbuf(r):
        return rows.at[pl.ds(lax.rem(r, 3) * 128, 128)]

    def drain(sem_):
        pltpu.make_async_copy(
            tp_in.at[pl.ds(0, 128)], rows.at[pl.ds(0, 128)], sem_).wait()

    for reg in range(2):
        regid = (2 * s + reg) * 2 + c
        pltpu.sync_copy(cnts.at[pl.ds(regid * 16, 16)], cv)
        nr = jnp.sum(jnp.where(lane == 0, cv[pl.ds(0, 16)], 0))
        rowbase = regid * L_ROWS

        def gpair(r):
            for h in range(2):
                pltpu.async_copy(
                    tp_in.at[sb.at[pl.ds(r * 128 + h * 64, 64)]],
                    rows.at[pl.ds(lax.rem(r, 3) * 128 + h * 64, 64)], sem)

        def gdrain():
            pltpu.make_async_copy(
                tp_in.at[pl.ds(0, 64)], rows.at[pl.ds(0, 64)], sem).wait()

        def sub(k, _):
            @pl.when(k * 16 < nr)
            def _():
                pltpu.sync_copy(
                    lsrc1.at[pl.ds(rowbase * 128 + k * 2048, 2048)], sb)
                pltpu.sync_copy(ldst.at[pl.ds(rowbase + k * 16, 16)], db)
                nrr = jnp.minimum(nr - k * 16, 16)
                gpair(0)

                @pl.when(nrr > 1)
                def _():
                    gpair(1)

                def inner(r, _):
                    gdrain()
                    gdrain()
                    pltpu.async_copy(
                        dbuf(r), acc.at[db.at[r]], sem2, add=True)

                    @pl.when(r >= 1)
                    def _():
                        drain(sem2)

                    @pl.when(r + 2 < nrr)
                    def _():
                        gpair(r + 2)
                    return 0

                lax.fori_loop(0, nrr, inner, 0)
                drain(sem2)
            return 0

        lax.fori_loop(0, L_ROWS // 16, sub, 0)
    plsc.subcore_barrier()

    def rows_out(j, _):
        g = j * NS + s

        @pl.when(g < ROW_BLOCKS)
        def _():
            lstart = jnp.minimum(g * 128, LAST_ROW_START)
            gstart = c * HALF + lstart
            pltpu.sync_copy(acc.at[pl.ds(lstart, 128)], rows.at[pl.ds(0, 128)])
            pltpu.sync_copy(win_h.at[pl.ds(gstart, 128)], wv)
            pltpu.sync_copy(wprod_h.at[pl.ds(gstart, 128)], wpv)

            def scale(r, _):
                idx = jnp.full((16,), r, _i32)
                w = plsc.load_gather(wv, [idx])
                wp = plsc.load_gather(wpv, [idx])
                for q in range(4):
                    sl = pl.ds(q * 16, 16)
                    sv = rows[r, sl]
                    rows[r + 128, sl] = sv * w
                    rows[r + 256, sl] = sv * wp
                return 0

            lax.fori_loop(0, 128, scale, 0)
            pltpu.sync_copy(rows.at[pl.ds(128, 128)], t_out.at[pl.ds(gstart, 128)])
            pltpu.sync_copy(rows.at[pl.ds(256, 128)], tp_out.at[pl.ds(gstart, 128)])
        return 0

    lax.fori_loop(0, 13, rows_out, 0)


@functools.partial(
    pl.kernel,
    out_type=jax.ShapeDtypeStruct((BATCH,), _f32),
    mesh=_mesh,
    compiler_params=pltpu.CompilerParams(use_tc_tiling_on_sc=False, needs_layout_passes=False),
    scratch_types=[
        pltpu.VMEM((128,), _i32),
        pltpu.VMEM((128,), _i32),
        pltpu.VMEM((128, D), _f32), pltpu.VMEM((128, D), _f32),
        pltpu.VMEM((128, D), _f32), pltpu.VMEM((128, D), _f32),
        pltpu.VMEM((128, D), _f32), pltpu.VMEM((128, D), _f32),
        pltpu.VMEM((128, D), _f32), pltpu.VMEM((128, D), _f32),
        pltpu.VMEM((128,), _f32),
        pltpu.SemaphoreType.DMA,
    ],
)
def _final(users_h, items_h, t0, t1, t2, t3, gamma_h,
           ub, vb, u0, u1, u2, u3, i0, i1, i2, i3, gb, sem):
    c = lax.axis_index("c")
    s = lax.axis_index("s")
    wid = c * NS + s

    def block(t, _):
        sb = wid * 4 + t
        pltpu.sync_copy(users_h.at[pl.ds(sb * 128, 128)], ub)
        pltpu.sync_copy(items_h.at[pl.ds(sb * 128, 128)], vb)
        for q in range(8):
            sl = pl.ds(q * 16, 16)
            vb[sl] = vb[sl] + NU
        cps = [
            pltpu.async_copy(t0.at[ub], u0, sem),
            pltpu.async_copy(t1.at[ub], u1, sem),
            pltpu.async_copy(t2.at[ub], u2, sem),
            pltpu.async_copy(t3.at[ub], u3, sem),
            pltpu.async_copy(t0.at[vb], i0, sem),
            pltpu.async_copy(t1.at[vb], i1, sem),
            pltpu.async_copy(t2.at[vb], i2, sem),
            pltpu.async_copy(t3.at[vb], i3, sem),
        ]
        for cp in cps:
            cp.wait()

        lane = lax.iota(_i32, 16)

        def dot(rb, _):
            res = jnp.zeros((16,), _f32)
            for rr in range(16):
                r = rb * 16 + rr
                acc = jnp.zeros((16,), _f32)
                for q in range(4):
                    sl = (r, pl.ds(q * 16, 16))
                    fu = (u0[sl] + u1[sl]) + (u2[sl] + u3[sl])
                    fi = (i0[sl] + i1[sl]) + (i2[sl] + i3[sl])
                    acc = acc + fu * fi
                res = res + jnp.where(lane == rr, jnp.sum(acc), 0.0)
            gb[pl.ds(rb * 16, 16)] = res * 0.0625
            return 0

        lax.fori_loop(0, 8, dot, 0)
        pltpu.sync_copy(gb, gamma_h.at[pl.ds(sb * 128, 128)])
        return 0

    lax.fori_loop(0, 4, block, 0)


def kernel(users, items, edge_index, user_emb, item_emb):
    srcp = jnp.pad(edge_index[0], (0, E_PAD - E),
                   constant_values=NN).reshape(ER_PAD, 128)
    dstp = jnp.pad(edge_index[1], (0, E_PAD - E),
                   constant_values=NN).reshape(ER_PAD, 128)
    t0 = jnp.concatenate([user_emb, item_emb], axis=0)
    t0 = jnp.pad(t0, ((0, NN_PAD - NN), (0, 0)))

    lsrc, ldst, cnts, deg_s, deg_d = _partition_edges(srcp, dstp)
    ldst2 = ldst.reshape(NREG * L_ROWS, 128)
    ds2 = deg_s.reshape(2, NN_PAD, 1)
    dd2 = deg_d.reshape(2, NN_PAD, 1)
    win, wprod, tp = _prep_tc(ds2[0], ds2[1], dd2[0], dd2[1], t0)
    win1 = win.reshape(NN_PAD)
    wprod1 = wprod.reshape(NN_PAD)
    t1, tp = _layer(tp, lsrc, ldst2, cnts, win1, wprod1)
    t2, tp = _layer(tp, lsrc, ldst2, cnts, win1, wprod1)
    t3, _ = _layer(tp, lsrc, ldst2, cnts, win1, wprod1)

    return _final(users, items, t0, t1, t2, t3)
